# baseline (speedup 1.0000x reference)
import functools

import jax
import jax.numpy as jnp
from jax import lax
from jax.experimental import pallas as pl
from jax.experimental.pallas import tpu as pltpu
from jax.experimental.pallas import tpu_sc as plsc

VOCAB = 100000
D = 64
B = 4096
S = 200

_INFO = plsc.get_sparse_core_info()
NC = _INFO.num_cores
NS = _INFO.num_subcores
NW = NC * NS

B_W = B // NW
NBUF = 4
GROUPS = B_W // NBUF


def _body(xf_hbm, xt_hbm, wm_hbm, tt_hbm, out_hbm, idxf_v, idxt_v, rows_v, sems):
    wid = lax.axis_index("s") * NC + lax.axis_index("c")
    base = pl.multiple_of(wid * B_W * S, B_W * S)

    pltpu.sync_copy(xf_hbm.at[pl.ds(base, B_W * S)], idxf_v)
    pltpu.sync_copy(xt_hbm.at[pl.ds(base, B_W * S)], idxt_v)

    def gather_a(g, b):
        off = (g * NBUF + b) * S
        return pltpu.make_async_copy(wm_hbm.at[idxf_v.at[pl.ds(off, S)]],
                                     rows_v.at[b], sems.at[b])

    def gather_b(g, b):
        off = (g * NBUF + b) * S
        return pltpu.make_async_copy(tt_hbm.at[idxt_v.at[pl.ds(off, S)]],
                                     rows_v.at[b], sems.at[b])

    def scatter_out(g, b):
        bi = wid * B_W + g * NBUF + b
        return pltpu.make_async_copy(rows_v.at[b],
                                     out_hbm.at[bi, :, pl.ds(0, D)],
                                     sems.at[b])

    def group_body(g, carry):
        for b in range(NBUF):
            @pl.when(g > 0)
            def _wait_prev():
                scatter_out(g - 1, b).wait()
            gather_a(g, b).start()
        for b in range(NBUF):
            gather_a(g, b).wait()
            gather_b(g, b).start(add=True)
        for b in range(NBUF):
            gather_b(g, b).wait()
            scatter_out(g, b).start()
        return carry

    lax.fori_loop(0, GROUPS, group_body, 0)
    for b in range(NBUF):
        scatter_out(GROUPS - 1, b).wait()


_BB = 16


def _slice_body(i_ref, o_ref):
    o_ref[...] = i_ref[:, :, :D]


def _slice_tc(out128):
    return pl.pallas_call(
        _slice_body,
        grid=(B // _BB,),
        in_specs=[pl.BlockSpec((_BB, S, 128), lambda i: (i, 0, 0))],
        out_specs=pl.BlockSpec((_BB, S, D), lambda i: (i, 0, 0)),
        out_shape=jax.ShapeDtypeStruct((B, S, D), jnp.float32),
    )(out128)


@jax.jit
def _dual_embed(xf, xt, wm, tt):
    mesh = plsc.VectorSubcoreMesh(core_axis_name="c", subcore_axis_name="s")
    f = functools.partial(
        pl.kernel,
        out_type=jax.ShapeDtypeStruct((B, S, 128), jnp.float32),
        mesh=mesh,
        scratch_types=[
            pltpu.VMEM((B_W * S,), jnp.int32),
            pltpu.VMEM((B_W * S,), jnp.int32),
            pltpu.VMEM((NBUF, S, D), jnp.float32),
            pltpu.SemaphoreType.DMA((NBUF,)),
        ],
        compiler_params=pltpu.CompilerParams(use_tc_tiling_on_sc=False),
    )(_body)
    return f(xf, xt, wm, tt)


def kernel(x_fix, x_train, word_mat, trained_table):
    b, s = x_fix.shape
    xf = x_fix.reshape(-1).astype(jnp.int32)
    xt = x_train.reshape(-1).astype(jnp.int32)
    out = _dual_embed(xf, xt, word_mat, trained_table)
    return _slice_tc(out)


# speedup vs baseline: 1.7710x; 1.7710x over previous
import functools

import jax
import jax.numpy as jnp
from jax import lax
from jax.experimental import pallas as pl
from jax.experimental.pallas import tpu as pltpu
from jax.experimental.pallas import tpu_sc as plsc

VOCAB = 100000
D = 64
B = 4096
S = 200

_INFO = plsc.get_sparse_core_info()
NC = _INFO.num_cores
NS = _INFO.num_subcores
NW = NC * NS

B_W = B // NW
NBUF = 4
GROUPS = B_W // NBUF


def _body(xf_hbm, xt_hbm, wm_hbm, tt_hbm, out_hbm, idxf_v, idxt_v, rows_v, sems):
    wid = lax.axis_index("s") * NC + lax.axis_index("c")
    base = pl.multiple_of(wid * B_W, B_W)

    pltpu.sync_copy(xf_hbm.at[pl.ds(base, B_W)], idxf_v)
    pltpu.sync_copy(xt_hbm.at[pl.ds(base, B_W)], idxt_v)

    def gather_a(g, b):
        return pltpu.make_async_copy(wm_hbm.at[idxf_v.at[g * NBUF + b]],
                                     rows_v.at[b], sems.at[b])

    def gather_b(g, b):
        return pltpu.make_async_copy(tt_hbm.at[idxt_v.at[g * NBUF + b]],
                                     rows_v.at[b], sems.at[b])

    def scatter_out(g, b):
        bi = wid * B_W + g * NBUF + b
        return pltpu.make_async_copy(rows_v.at[b],
                                     out_hbm.at[bi, :, pl.ds(0, D)],
                                     sems.at[b])

    def group_body(g, carry):
        for b in range(NBUF):
            @pl.when(g > 0)
            def _wait_prev():
                scatter_out(g - 1, b).wait()
            gather_a(g, b).start()
        for b in range(NBUF):
            gather_a(g, b).wait()
            gather_b(g, b).start(add=True)
        for b in range(NBUF):
            gather_b(g, b).wait()
            scatter_out(g, b).start()
        return carry

    lax.fori_loop(0, GROUPS, group_body, 0)
    for b in range(NBUF):
        scatter_out(GROUPS - 1, b).wait()


@jax.jit
def _dual_embed(xf, xt, wm, tt):
    mesh = plsc.VectorSubcoreMesh(core_axis_name="c", subcore_axis_name="s")
    f = functools.partial(
        pl.kernel,
        out_type=jax.ShapeDtypeStruct((B, S, 128), jnp.float32),
        mesh=mesh,
        scratch_types=[
            pltpu.VMEM((B_W, S), jnp.int32),
            pltpu.VMEM((B_W, S), jnp.int32),
            pltpu.VMEM((NBUF, S, D), jnp.float32),
            pltpu.SemaphoreType.DMA((NBUF,)),
        ],
        compiler_params=pltpu.CompilerParams(use_tc_tiling_on_sc=False),
    )(_body)
    return f(xf, xt, wm, tt)


def kernel(x_fix, x_train, word_mat, trained_table):
    out = _dual_embed(x_fix.astype(jnp.int32), x_train.astype(jnp.int32),
                      word_mat, trained_table)
    return out[:, :, :D]


# speedup vs baseline: 1.7788x; 1.0044x over previous
import functools

import jax
import jax.numpy as jnp
from jax import lax
from jax.experimental import pallas as pl
from jax.experimental.pallas import tpu as pltpu
from jax.experimental.pallas import tpu_sc as plsc

VOCAB = 100000
D = 64
B = 4096
S = 200

_INFO = plsc.get_sparse_core_info()
NC = _INFO.num_cores
NS = _INFO.num_subcores
NW = NC * NS

B_W = B // NW
NBUF = 4
GROUPS = B_W // NBUF


def _body(xf_hbm, xt_hbm, wm_hbm, tt_hbm, out_hbm, idxf_v, idxt_v, rows_v, sems):
    wid = lax.axis_index("s") * NC + lax.axis_index("c")
    base = pl.multiple_of(wid * B_W, B_W)

    pltpu.sync_copy(xf_hbm.at[pl.ds(base, B_W)], idxf_v)
    pltpu.sync_copy(xt_hbm.at[pl.ds(base, B_W)], idxt_v)

    H0 = 104
    H1 = S - H0

    def _half(b):
        k, h = b // 2, b % 2
        return k, (0 if h == 0 else H0), (H0 if h == 0 else H1)

    def gather_a(g, b):
        k, off, sz = _half(b)
        return pltpu.make_async_copy(
            wm_hbm.at[idxf_v.at[g * NBUF + k, pl.ds(off, sz)]],
            rows_v.at[b, pl.ds(0, sz)], sems.at[b])

    def gather_b(g, b):
        k, off, sz = _half(b)
        return pltpu.make_async_copy(
            tt_hbm.at[idxt_v.at[g * NBUF + k, pl.ds(off, sz)]],
            rows_v.at[b, pl.ds(0, sz)], sems.at[b])

    def scatter_out(g, b):
        k, off, sz = _half(b)
        bi = wid * B_W + g * NBUF + k
        return pltpu.make_async_copy(rows_v.at[b, pl.ds(0, sz)],
                                     out_hbm.at[bi, pl.ds(off, sz),
                                                pl.ds(0, D)],
                                     sems.at[b])

    def group_body(g, carry):
        for b in range(2 * NBUF):
            @pl.when(g > 0)
            def _wait_prev():
                scatter_out(g - 1, b).wait()
            gather_a(g, b).start()
        for b in range(2 * NBUF):
            gather_a(g, b).wait()
            gather_b(g, b).start(add=True)
        for b in range(2 * NBUF):
            gather_b(g, b).wait()
            scatter_out(g, b).start()
        return carry

    lax.fori_loop(0, GROUPS, group_body, 0)
    for b in range(2 * NBUF):
        scatter_out(GROUPS - 1, b).wait()


@jax.jit
def _dual_embed(xf, xt, wm, tt):
    mesh = plsc.VectorSubcoreMesh(core_axis_name="c", subcore_axis_name="s")
    f = functools.partial(
        pl.kernel,
        out_type=jax.ShapeDtypeStruct((B, S, 128), jnp.float32),
        mesh=mesh,
        scratch_types=[
            pltpu.VMEM((B_W, S), jnp.int32),
            pltpu.VMEM((B_W, S), jnp.int32),
            pltpu.VMEM((2 * NBUF, 104, D), jnp.float32),
            pltpu.SemaphoreType.DMA((2 * NBUF,)),
        ],
        compiler_params=pltpu.CompilerParams(use_tc_tiling_on_sc=False),
    )(_body)
    return f(xf, xt, wm, tt)


def kernel(x_fix, x_train, word_mat, trained_table):
    out = _dual_embed(x_fix.astype(jnp.int32), x_train.astype(jnp.int32),
                      word_mat, trained_table)
    return out[:, :, :D]
